# baseline (speedup 1.0000x reference)
import functools

import jax
import jax.numpy as jnp
from jax import lax
from jax.experimental import pallas as pl
from jax.experimental.pallas import tpu as pltpu
from jax.experimental.pallas import tpu_sc as plsc

_N = 10000
_E = 320000
_DIN = 128
_DHID = 128
_DOUT = 64

_NC = 2
_NS = 16
_NW = _NC * _NS
_EPW = _E // _NW
_K = 128
_KR = 125
_PCH = 40
_PH = 2
_NCHUNK = _PH * _PCH
_RPT = 640
_NPAD = _NS * _RPT
_DRPT = 640
_DPAD = _NS * _DRPT

_BLK = 1000


def _sc_mesh():
    return plsc.VectorSubcoreMesh(core_axis_name="c", subcore_axis_name="s")


@functools.partial(
    pl.kernel,
    out_type=jax.ShapeDtypeStruct((_NC, _DPAD), jnp.float32),
    mesh=_sc_mesh(),
    scratch_types=[
        pltpu.VMEM_SHARED((_DPAD,), jnp.float32),
        pltpu.VMEM((_NCHUNK, _K), jnp.int32),
        pltpu.VMEM((128,), jnp.float32),
    ],
)
def _deg_sc(dst_hbm, deg_hbm, deg_sp, idx_v, buf_v):
    c = lax.axis_index("c")
    s = lax.axis_index("s")
    w = c * _NS + s
    pltpu.sync_copy(dst_hbm.at[w], idx_v)

    def _fill(val):
        def body(j, carry):
            buf_v[pl.ds(j * 16, 16)] = jnp.full((16,), val, jnp.float32)
            return carry
        lax.fori_loop(0, 8, body, 0)

    _fill(0.0)
    base = s * _DRPT
    for t in range(_DRPT // 128):
        pltpu.sync_copy(buf_v, deg_sp.at[pl.ds(base + t * 128, 128)])
    _fill(1.0)
    plsc.subcore_barrier()

    def chunk(j, carry):
        pltpu.sync_copy(buf_v.at[pl.ds(0, _K)], deg_sp.at[idx_v.at[j]], add=True)
        return carry

    lax.fori_loop(0, _NCHUNK, chunk, 0)
    plsc.subcore_barrier()
    pltpu.sync_copy(deg_sp.at[pl.ds(base, _DRPT)], deg_hbm.at[c, pl.ds(base, _DRPT)])


def _make_agg(d_feat):
    @functools.partial(
        pl.kernel,
        out_type=jax.ShapeDtypeStruct((_NC, _NPAD, d_feat), jnp.float32),
        mesh=_sc_mesh(),
        scratch_types=[
            pltpu.VMEM_SHARED((_NPAD, d_feat), jnp.float32),
            pltpu.VMEM((_PCH, _K), jnp.int32),
            pltpu.VMEM((_PCH, _K), jnp.int32),
            pltpu.VMEM((_K, d_feat), jnp.float32),
            pltpu.VMEM((_K, d_feat), jnp.float32),
            pltpu.SemaphoreType.DMA,
            pltpu.SemaphoreType.DMA,
        ],
    )
    def agg(y_hbm, src_hbm, dst_hbm, acc_hbm,
            acc_sp, src_v, dst_v, buf0, buf1, sem0, sem1):
        c = lax.axis_index("c")
        s = lax.axis_index("s")
        w = c * _NS + s

        def drain0():
            pltpu.make_async_copy(y_hbm.at[pl.ds(0, _K)], buf0, sem0).wait()

        pltpu.sync_copy(src_hbm.at[w, 0], src_v)
        pltpu.sync_copy(dst_hbm.at[w, 0], dst_v)
        pltpu.async_copy(y_hbm.at[src_v.at[0]], buf0, sem0)

        def zrow(i, carry):
            def zcol(j, inner):
                buf1[i, pl.ds(j * 16, 16)] = jnp.zeros((16,), jnp.float32)
                return inner
            return lax.fori_loop(0, d_feat // 16, zcol, carry)

        lax.fori_loop(0, _K, zrow, 0)
        base = s * _RPT
        for t in range(_RPT // _K):
            pltpu.sync_copy(buf1, acc_sp.at[pl.ds(base + t * _K, _K)])
        plsc.subcore_barrier()

        for p in range(_PH):
            def pair(i, carry):
                g = 2 * i
                drain0()
                d1 = pltpu.async_copy(y_hbm.at[src_v.at[g + 1]], buf1, sem1)
                pltpu.sync_copy(buf0, acc_sp.at[dst_v.at[g]], add=True)
                d1.wait()
                pltpu.async_copy(y_hbm.at[src_v.at[g + 2]], buf0, sem0)
                pltpu.sync_copy(buf1, acc_sp.at[dst_v.at[g + 1]], add=True)
                return carry

            lax.fori_loop(0, (_PCH - 2) // 2, pair, 0)
            drain0()
            dl = pltpu.async_copy(y_hbm.at[src_v.at[_PCH - 1]], buf1, sem1)
            pltpu.sync_copy(buf0, acc_sp.at[dst_v.at[_PCH - 2]], add=True)
            dl.wait()
            if p + 1 < _PH:
                pltpu.sync_copy(src_hbm.at[w, p + 1], src_v)
                pltpu.sync_copy(buf1, acc_sp.at[dst_v.at[_PCH - 1]], add=True)
                pltpu.sync_copy(dst_hbm.at[w, p + 1], dst_v)
                pltpu.async_copy(y_hbm.at[src_v.at[0]], buf0, sem0)
            else:
                pltpu.sync_copy(buf1, acc_sp.at[dst_v.at[_PCH - 1]], add=True)
        plsc.subcore_barrier()

        pltpu.sync_copy(acc_sp.at[pl.ds(base, _RPT)],
                        acc_hbm.at[c, pl.ds(base, _RPT)])

    return agg


_agg_hid = _make_agg(_DHID)


def _dinv_from(dt_blk):
    return lax.rsqrt(dt_blk[:, 0:1] + dt_blk[:, 1:2] + 1.0)


def _tc_mm_body(x_ref, w_ref, u_ref):
    u_ref[...] = jnp.dot(x_ref[...], w_ref[...],
                         preferred_element_type=jnp.float32)


def _tc_mm(x, w1):
    return pl.pallas_call(
        _tc_mm_body,
        grid=(_N // _BLK,),
        in_specs=[
            pl.BlockSpec((_BLK, _DIN), lambda r: (r, 0)),
            pl.BlockSpec((_DIN, _DHID), lambda r: (0, 0)),
        ],
        out_specs=pl.BlockSpec((_BLK, _DHID), lambda r: (r, 0)),
        out_shape=jax.ShapeDtypeStruct((_N, _DHID), jnp.float32),
    )(x, w1)


def _tc_scale_body(u_ref, dt_ref, y_ref):
    y_ref[...] = u_ref[...] * _dinv_from(dt_ref[...])


def _tc_scale(u, deg_t):
    return pl.pallas_call(
        _tc_scale_body,
        grid=(_N // _BLK,),
        in_specs=[
            pl.BlockSpec((_BLK, _DHID), lambda r: (r, 0)),
            pl.BlockSpec((_BLK, 2), lambda r: (r, 0)),
        ],
        out_specs=pl.BlockSpec((_BLK, _DHID), lambda r: (r, 0)),
        out_shape=jax.ShapeDtypeStruct((_N, _DHID), jnp.float32),
    )(u, deg_t)


def _tc2_body(ap_ref, aq_ref, y1_ref, dt_ref, b1_ref, yh_ref):
    dinv = _dinv_from(dt_ref[...])
    h = (ap_ref[0] + aq_ref[0] + y1_ref[...]) * dinv + b1_ref[...]
    h = jnp.maximum(h, 0.0)
    yh_ref[...] = h * dinv


def _tc2(acc1, y1, deg_t, b1):
    return pl.pallas_call(
        _tc2_body,
        grid=(_N // _BLK,),
        in_specs=[
            pl.BlockSpec((1, _BLK, _DHID), lambda r: (0, r, 0)),
            pl.BlockSpec((1, _BLK, _DHID), lambda r: (1, r, 0)),
            pl.BlockSpec((_BLK, _DHID), lambda r: (r, 0)),
            pl.BlockSpec((_BLK, 2), lambda r: (r, 0)),
            pl.BlockSpec((1, _DHID), lambda r: (0, 0)),
        ],
        out_specs=pl.BlockSpec((_BLK, _DHID), lambda r: (r, 0)),
        out_shape=jax.ShapeDtypeStruct((_N, _DHID), jnp.float32),
    )(acc1, acc1, y1, deg_t, b1)


def _tc3_body(ap_ref, aq_ref, yh_ref, dt_ref, b2_ref, w2_ref, o_ref):
    dinv = _dinv_from(dt_ref[...])
    z = (ap_ref[0] + aq_ref[0] + yh_ref[...]) * dinv
    o = jnp.dot(z, w2_ref[...], preferred_element_type=jnp.float32) + b2_ref[...]
    m = jnp.max(o, axis=1, keepdims=True)
    lse = jnp.log(jnp.sum(jnp.exp(o - m), axis=1, keepdims=True)) + m
    o_ref[...] = o - lse


def _tc3(acc2, yh, deg_t, b2, w2):
    return pl.pallas_call(
        _tc3_body,
        grid=(_N // _BLK,),
        in_specs=[
            pl.BlockSpec((1, _BLK, _DHID), lambda r: (0, r, 0)),
            pl.BlockSpec((1, _BLK, _DHID), lambda r: (1, r, 0)),
            pl.BlockSpec((_BLK, _DHID), lambda r: (r, 0)),
            pl.BlockSpec((_BLK, 2), lambda r: (r, 0)),
            pl.BlockSpec((1, _DOUT), lambda r: (0, 0)),
            pl.BlockSpec((_DHID, _DOUT), lambda r: (0, 0)),
        ],
        out_specs=pl.BlockSpec((_BLK, _DOUT), lambda r: (r, 0)),
        out_shape=jax.ShapeDtypeStruct((_N, _DOUT), jnp.float32),
    )(acc2, acc2, yh, deg_t, b2, w2)


def kernel(x, edge_index, W1, b1, W2, b2):
    srcr = edge_index[0].astype(jnp.int32).reshape(_NW, _NCHUNK, _KR)
    dstr = edge_index[1].astype(jnp.int32).reshape(_NW, _NCHUNK, _KR)
    ch = jnp.arange(_NCHUNK, dtype=jnp.int32)[None, :, None]
    three = jnp.arange(3, dtype=jnp.int32)[None, None, :]
    src_pad = jnp.broadcast_to((ch * 125 + three * 41) % _N, (_NW, _NCHUNK, 3))
    dst_pad = jnp.broadcast_to(_N + (ch * 3 + three) % (_NPAD - _N),
                               (_NW, _NCHUNK, 3))
    src = jnp.concatenate([srcr, src_pad], axis=2)
    dst = jnp.concatenate([dstr, dst_pad], axis=2)
    deg2 = _deg_sc(dst)
    u = _tc_mm(x, W1)
    src = src.reshape(_NW, _PH, _PCH, _K)
    dst = dst.reshape(_NW, _PH, _PCH, _K)
    deg_t = deg2[:, :_N].T
    y1 = _tc_scale(u, deg_t)
    acc1 = _agg_hid(y1, src, dst)
    yh = _tc2(acc1, y1, deg_t, b1.reshape(1, _DHID))
    acc2 = _agg_hid(yh, src, dst)
    return _tc3(acc2, yh, deg_t, b2.reshape(1, _DOUT), W2)


# speedup vs baseline: 1.0445x; 1.0445x over previous
import functools

import jax
import jax.numpy as jnp
from jax import lax
from jax.experimental import pallas as pl
from jax.experimental.pallas import tpu as pltpu
from jax.experimental.pallas import tpu_sc as plsc

_N = 10000
_E = 320000
_DIN = 128
_DHID = 128
_DOUT = 64

_NC = 2
_NS = 16
_NW = _NC * _NS
_EPW = _E // _NW
_K = 128
_NCH = _E // _K
_CPW = _NCH // _NW
_TW0 = _NW - (_NCH - _NW * _CPW)
_P0 = 40
_P1 = _CPW - _P0
_RPT = 640
_NPAD = _NS * _RPT
_DRPT = 640
_DPAD = _NS * _DRPT

_BLK = 1000


def _sc_mesh():
    return plsc.VectorSubcoreMesh(core_axis_name="c", subcore_axis_name="s")


@functools.partial(
    pl.kernel,
    out_type=jax.ShapeDtypeStruct((_NC, _DPAD), jnp.float32),
    mesh=_sc_mesh(),
    scratch_types=[
        pltpu.VMEM_SHARED((_DPAD,), jnp.float32),
        pltpu.VMEM((_CPW, _K), jnp.int32),
        pltpu.VMEM((128,), jnp.float32),
    ],
)
def _deg_sc(dst_hbm, dstt_hbm, deg_hbm, deg_sp, idx_v, buf_v):
    c = lax.axis_index("c")
    s = lax.axis_index("s")
    w = c * _NS + s
    pltpu.sync_copy(dst_hbm.at[w], idx_v)

    def _fill(val):
        def body(j, carry):
            buf_v[pl.ds(j * 16, 16)] = jnp.full((16,), val, jnp.float32)
            return carry
        lax.fori_loop(0, 8, body, 0)

    _fill(0.0)
    base = s * _DRPT
    for t in range(_DRPT // 128):
        pltpu.sync_copy(buf_v, deg_sp.at[pl.ds(base + t * 128, 128)])
    _fill(1.0)
    plsc.subcore_barrier()

    def chunk(j, carry):
        pltpu.sync_copy(buf_v.at[pl.ds(0, _K)], deg_sp.at[idx_v.at[j]], add=True)
        return carry

    lax.fori_loop(0, _CPW, chunk, 0)

    @pl.when(w >= _TW0)
    def _tail():
        pltpu.sync_copy(dstt_hbm.at[w - _TW0], idx_v.at[pl.ds(0, 1)])
        pltpu.sync_copy(buf_v, deg_sp.at[idx_v.at[0]], add=True)

    plsc.subcore_barrier()
    pltpu.sync_copy(deg_sp.at[pl.ds(base, _DRPT)], deg_hbm.at[c, pl.ds(base, _DRPT)])


def _make_agg(d_feat):
    @functools.partial(
        pl.kernel,
        out_type=jax.ShapeDtypeStruct((_NC, _NPAD, d_feat), jnp.float32),
        mesh=_sc_mesh(),
        scratch_types=[
            pltpu.VMEM_SHARED((_NPAD, d_feat), jnp.float32),
            pltpu.VMEM((_P0, _K), jnp.int32),
            pltpu.VMEM((_P0, _K), jnp.int32),
            pltpu.VMEM((_K, d_feat), jnp.float32),
            pltpu.VMEM((_K, d_feat), jnp.float32),
            pltpu.SemaphoreType.DMA,
            pltpu.SemaphoreType.DMA,
        ],
    )
    def agg(y_hbm, src_hbm, dst_hbm, srct_hbm, dstt_hbm, acc_hbm,
            acc_sp, src_v, dst_v, buf0, buf1, sem0, sem1):
        c = lax.axis_index("c")
        s = lax.axis_index("s")
        w = c * _NS + s

        def drain0():
            pltpu.make_async_copy(y_hbm.at[pl.ds(0, _K)], buf0, sem0).wait()

        pltpu.sync_copy(src_hbm.at[w, pl.ds(0, _P0)], src_v)
        pltpu.sync_copy(dst_hbm.at[w, pl.ds(0, _P0)], dst_v)
        pltpu.async_copy(y_hbm.at[src_v.at[0]], buf0, sem0)

        def zrow(i, carry):
            def zcol(j, inner):
                buf1[i, pl.ds(j * 16, 16)] = jnp.zeros((16,), jnp.float32)
                return inner
            return lax.fori_loop(0, d_feat // 16, zcol, carry)

        lax.fori_loop(0, _K, zrow, 0)
        base = s * _RPT
        for t in range(_RPT // _K):
            pltpu.sync_copy(buf1, acc_sp.at[pl.ds(base + t * _K, _K)])
        plsc.subcore_barrier()

        def pair(i, carry):
            g = 2 * i
            drain0()
            d1 = pltpu.async_copy(y_hbm.at[src_v.at[g + 1]], buf1, sem1)
            pltpu.sync_copy(buf0, acc_sp.at[dst_v.at[g]], add=True)
            d1.wait()
            pltpu.async_copy(y_hbm.at[src_v.at[g + 2]], buf0, sem0)
            pltpu.sync_copy(buf1, acc_sp.at[dst_v.at[g + 1]], add=True)
            return carry

        lax.fori_loop(0, (_P0 - 2) // 2, pair, 0)
        drain0()
        dl = pltpu.async_copy(y_hbm.at[src_v.at[_P0 - 1]], buf1, sem1)
        pltpu.sync_copy(buf0, acc_sp.at[dst_v.at[_P0 - 2]], add=True)
        dl.wait()
        pltpu.sync_copy(src_hbm.at[w, pl.ds(_P0, _P1)],
                        src_v.at[pl.ds(0, _P1)])
        pltpu.sync_copy(buf1, acc_sp.at[dst_v.at[_P0 - 1]], add=True)
        pltpu.sync_copy(dst_hbm.at[w, pl.ds(_P0, _P1)],
                        dst_v.at[pl.ds(0, _P1)])
        pltpu.async_copy(y_hbm.at[src_v.at[0]], buf0, sem0)

        lax.fori_loop(0, (_P1 - 2) // 2, pair, 0)
        drain0()
        dl = pltpu.async_copy(y_hbm.at[src_v.at[_P1 - 1]], buf1, sem1)
        pltpu.sync_copy(buf0, acc_sp.at[dst_v.at[_P1 - 2]], add=True)
        dl.wait()
        pltpu.sync_copy(buf1, acc_sp.at[dst_v.at[_P1 - 1]], add=True)

        @pl.when(w >= _TW0)
        def _tail():
            t = w - _TW0
            pltpu.sync_copy(srct_hbm.at[t], src_v.at[pl.ds(0, 1)])
            pltpu.sync_copy(dstt_hbm.at[t], dst_v.at[pl.ds(0, 1)])
            pltpu.sync_copy(y_hbm.at[src_v.at[0]], buf0)
            pltpu.sync_copy(buf0, acc_sp.at[dst_v.at[0]], add=True)

        plsc.subcore_barrier()

        pltpu.sync_copy(acc_sp.at[pl.ds(base, _RPT)],
                        acc_hbm.at[c, pl.ds(base, _RPT)])

    return agg


_agg_hid = _make_agg(_DHID)


def _dinv_from(dt_blk):
    return lax.rsqrt(dt_blk[:, 0:1] + dt_blk[:, 1:2] + 1.0)


def _tc1_body(x_ref, w_ref, dt_ref, y_ref):
    dinv = _dinv_from(dt_ref[...])
    y_ref[...] = jnp.dot(x_ref[...], w_ref[...],
                         preferred_element_type=jnp.float32) * dinv


def _tc1(x, w1, deg_t):
    return pl.pallas_call(
        _tc1_body,
        grid=(_N // _BLK,),
        in_specs=[
            pl.BlockSpec((_BLK, _DIN), lambda r: (r, 0)),
            pl.BlockSpec((_DIN, _DHID), lambda r: (0, 0)),
            pl.BlockSpec((_BLK, 2), lambda r: (r, 0)),
        ],
        out_specs=pl.BlockSpec((_BLK, _DHID), lambda r: (r, 0)),
        out_shape=jax.ShapeDtypeStruct((_N, _DHID), jnp.float32),
    )(x, w1, deg_t)


def _tc2_body(ap_ref, aq_ref, y1_ref, dt_ref, b1_ref, yh_ref):
    dinv = _dinv_from(dt_ref[...])
    h = (ap_ref[0] + aq_ref[0] + y1_ref[...]) * dinv + b1_ref[...]
    h = jnp.maximum(h, 0.0)
    yh_ref[...] = h * dinv


def _tc2(acc1, y1, deg_t, b1):
    return pl.pallas_call(
        _tc2_body,
        grid=(_N // _BLK,),
        in_specs=[
            pl.BlockSpec((1, _BLK, _DHID), lambda r: (0, r, 0)),
            pl.BlockSpec((1, _BLK, _DHID), lambda r: (1, r, 0)),
            pl.BlockSpec((_BLK, _DHID), lambda r: (r, 0)),
            pl.BlockSpec((_BLK, 2), lambda r: (r, 0)),
            pl.BlockSpec((1, _DHID), lambda r: (0, 0)),
        ],
        out_specs=pl.BlockSpec((_BLK, _DHID), lambda r: (r, 0)),
        out_shape=jax.ShapeDtypeStruct((_N, _DHID), jnp.float32),
    )(acc1, acc1, y1, deg_t, b1)


def _tc3_body(ap_ref, aq_ref, yh_ref, dt_ref, b2_ref, w2_ref, o_ref):
    dinv = _dinv_from(dt_ref[...])
    z = (ap_ref[0] + aq_ref[0] + yh_ref[...]) * dinv
    o = jnp.dot(z, w2_ref[...], preferred_element_type=jnp.float32) + b2_ref[...]
    m = jnp.max(o, axis=1, keepdims=True)
    lse = jnp.log(jnp.sum(jnp.exp(o - m), axis=1, keepdims=True)) + m
    o_ref[...] = o - lse


def _tc3(acc2, yh, deg_t, b2, w2):
    return pl.pallas_call(
        _tc3_body,
        grid=(_N // _BLK,),
        in_specs=[
            pl.BlockSpec((1, _BLK, _DHID), lambda r: (0, r, 0)),
            pl.BlockSpec((1, _BLK, _DHID), lambda r: (1, r, 0)),
            pl.BlockSpec((_BLK, _DHID), lambda r: (r, 0)),
            pl.BlockSpec((_BLK, 2), lambda r: (r, 0)),
            pl.BlockSpec((1, _DOUT), lambda r: (0, 0)),
            pl.BlockSpec((_DHID, _DOUT), lambda r: (0, 0)),
        ],
        out_specs=pl.BlockSpec((_BLK, _DOUT), lambda r: (r, 0)),
        out_shape=jax.ShapeDtypeStruct((_N, _DOUT), jnp.float32),
    )(acc2, acc2, yh, deg_t, b2, w2)


def kernel(x, edge_index, W1, b1, W2, b2):
    e0 = edge_index[0].astype(jnp.int32)
    e1 = edge_index[1].astype(jnp.int32)
    nmain = _NW * _CPW * _K
    src = e0[:nmain].reshape(_NW, _CPW, _K)
    dst = e1[:nmain].reshape(_NW, _CPW, _K)
    srct = e0[nmain:].reshape(_NCH - _NW * _CPW, 1, _K)
    dstt = e1[nmain:].reshape(_NCH - _NW * _CPW, 1, _K)
    deg2 = _deg_sc(dst, dstt)
    deg_t = deg2[:, :_N].T
    y1 = _tc1(x, W1, deg_t)
    acc1 = _agg_hid(y1, src, dst, srct, dstt)
    yh = _tc2(acc1, y1, deg_t, b1.reshape(1, _DHID))
    acc2 = _agg_hid(yh, src, dst, srct, dstt)
    return _tc3(acc2, yh, deg_t, b2.reshape(1, _DOUT), W2)
